# baseline (device time: 30982 ns/iter reference)
import jax
import jax.numpy as jnp
from jax import lax
from jax.experimental import pallas as pl
from jax.experimental.pallas import tpu as pltpu

M = 1024
N_LOCAL = 8192
K = 32
ROW_BLK = 128
N_DEV = 16
N_GRP = 8
N_GROUP = 128
CAND_T = 7
NEG_INF = float("-inf")


def _rev_lanes(x):
    w = x.shape[1]
    idx = lax.broadcasted_iota(jnp.int32, x.shape, 1)
    s = w // 2
    while s >= 1:
        left = jnp.roll(x, -s, axis=1)
        right = jnp.roll(x, s, axis=1)
        x = jnp.where((idx & s) == 0, left, right)
        s //= 2
    return x


def _merge_sorted_desc(a, b):
    c = jnp.concatenate([a, _rev_lanes(b)], axis=1)
    idx = lax.broadcasted_iota(jnp.int32, c.shape, 1)
    s = K
    while s >= 1:
        left = jnp.roll(c, -s, axis=1)
        right = jnp.roll(c, s, axis=1)
        up = (idx & s) == 0
        partner = jnp.where(up, left, right)
        c = jnp.where(up, jnp.maximum(c, partner), jnp.minimum(c, partner))
        s //= 2
    return c[:, :K]


def _topk_desc(vals_list, n_rows, k):
    out = jnp.full((n_rows, k), NEG_INF, dtype=jnp.float32)
    col = lax.broadcasted_iota(jnp.int32, (n_rows, k), 1)
    arrs = list(vals_list)
    for i in range(k):
        m = arrs[0].max(axis=1, keepdims=True)
        for a in arrs[1:]:
            m = jnp.maximum(m, a.max(axis=1, keepdims=True))
        out = jnp.where(col == i, m, out)
        arrs = [jnp.where(a == m, NEG_INF, a) for a in arrs]
    return out


def _chunk_topk(xv):
    x = xv
    cands = []
    for t in range(CAND_T):
        a = x
        w = N_LOCAL
        while w > N_GROUP:
            w //= 2
            a = jnp.maximum(a[:, :w], a[:, w:])
        cands.append(a)
        if t < CAND_T - 1:
            b = jnp.concatenate([a] * (N_LOCAL // N_GROUP), axis=1)
            x = jnp.where(x == b, NEG_INF, x)

    F = cands[0]
    d = jnp.zeros(F.shape, jnp.int32)
    out = jnp.full((ROW_BLK, K), NEG_INF, dtype=jnp.float32)
    col = lax.broadcasted_iota(jnp.int32, (ROW_BLK, K), 1)
    for i in range(K):
        m = F.max(axis=1, keepdims=True)
        out = jnp.where(col == i, m, out)
        adv = F == m
        d = d + adv.astype(jnp.int32)
        nxt = jnp.full_like(F, NEG_INF)
        for t in range(CAND_T - 1, 0, -1):
            nxt = jnp.where(d == t, cands[t], nxt)
        F = jnp.where(adv, nxt, F)
    return out


def kernel(x):
    def body(x_hbm, o_ref, xv_ref, lbuf_ref, in0_ref, in1_ref, dma_sem,
             bc_send, bc_recv):
        mx = lax.axis_index("x")
        my = lax.axis_index("y")
        mz = lax.axis_index("z")
        g = mx * 4 + mz
        my_id = mx * 8 + my * 4 + mz

        cp = pltpu.make_async_copy(
            x_hbm.at[pl.ds(g * ROW_BLK, ROW_BLK), :], xv_ref, dma_sem
        )
        cp.start()

        bsem = pltpu.get_barrier_semaphore()
        for t in range(N_DEV):
            tx, ty, tz = t // 8, (t // 4) % 2, t % 4

            @pl.when(my_id != t)
            def _():
                pl.semaphore_signal(bsem, inc=1, device_id=(tx, ty, tz),
                                    device_id_type=pl.DeviceIdType.MESH)

        pl.semaphore_wait(bsem, N_DEV - 1)
        cp.wait()

        lbuf_ref[...] = _chunk_topk(xv_ref[...])

        @pl.when(my == 0)
        def _():
            in0_ref[pl.ds(g * ROW_BLK, ROW_BLK), :] = lbuf_ref[...]

        @pl.when(my == 1)
        def _():
            in1_ref[pl.ds(g * ROW_BLK, ROW_BLK), :] = lbuf_ref[...]

        for t in range(N_DEV):
            tx, ty, tz = t // 8, (t // 4) % 2, t % 4

            @pl.when(my_id != t)
            def _():
                @pl.when(my == 0)
                def _():
                    snd = pltpu.make_async_remote_copy(
                        src_ref=lbuf_ref,
                        dst_ref=in0_ref.at[pl.ds(g * ROW_BLK, ROW_BLK), :],
                        send_sem=bc_send.at[t],
                        recv_sem=bc_recv.at[my_id],
                        device_id=(tx, ty, tz),
                        device_id_type=pl.DeviceIdType.MESH,
                    )
                    snd.start()

                @pl.when(my == 1)
                def _():
                    snd = pltpu.make_async_remote_copy(
                        src_ref=lbuf_ref,
                        dst_ref=in1_ref.at[pl.ds(g * ROW_BLK, ROW_BLK), :],
                        send_sem=bc_send.at[t],
                        recv_sem=bc_recv.at[my_id],
                        device_id=(tx, ty, tz),
                        device_id_type=pl.DeviceIdType.MESH,
                    )
                    snd.start()

        for t in range(N_DEV):
            tx, ty, tz = t // 8, (t // 4) % 2, t % 4

            @pl.when(my_id != t)
            def _():
                rcv = pltpu.make_async_remote_copy(
                    src_ref=lbuf_ref,
                    dst_ref=in0_ref.at[pl.ds(0, ROW_BLK), :],
                    send_sem=bc_send.at[t],
                    recv_sem=bc_recv.at[t],
                    device_id=(tx, ty, tz),
                    device_id_type=pl.DeviceIdType.MESH,
                )
                rcv.wait_recv()

        o_ref[...] = _merge_sorted_desc(in0_ref[...], in1_ref[...])

        for t in range(N_DEV):
            tx, ty, tz = t // 8, (t // 4) % 2, t % 4

            @pl.when(my_id != t)
            def _():
                snt = pltpu.make_async_remote_copy(
                    src_ref=lbuf_ref,
                    dst_ref=in0_ref.at[pl.ds(0, ROW_BLK), :],
                    send_sem=bc_send.at[t],
                    recv_sem=bc_recv.at[t],
                    device_id=(tx, ty, tz),
                    device_id_type=pl.DeviceIdType.MESH,
                )
                snt.wait_send()

    return pl.pallas_call(
        body,
        out_shape=jax.ShapeDtypeStruct((M, K), jnp.float32),
        in_specs=[pl.BlockSpec(memory_space=pl.ANY)],
        out_specs=pl.BlockSpec(memory_space=pltpu.VMEM),
        scratch_shapes=[
            pltpu.VMEM((ROW_BLK, N_LOCAL), jnp.float32),
            pltpu.VMEM((ROW_BLK, K), jnp.float32),
            pltpu.VMEM((M, K), jnp.float32),
            pltpu.VMEM((M, K), jnp.float32),
            pltpu.SemaphoreType.DMA,
            pltpu.SemaphoreType.DMA((N_DEV,)),
            pltpu.SemaphoreType.DMA((N_DEV,)),
        ],
        compiler_params=pltpu.CompilerParams(
            collective_id=0, vmem_limit_bytes=100 * 1024 * 1024
        ),
    )(x)


# device time: 22927 ns/iter; 1.3513x vs baseline; 1.3513x over previous
import jax
import jax.numpy as jnp
from jax import lax
from jax.experimental import pallas as pl
from jax.experimental.pallas import tpu as pltpu

M = 1024
N_LOCAL = 8192
K = 32
ROW_BLK = 128
N_GRP = 8
N_GROUP = 128
CAND_T = 7
NEG_INF = float("-inf")


def _rev_lanes(x):
    w = x.shape[1]
    idx = lax.broadcasted_iota(jnp.int32, x.shape, 1)
    s = w // 2
    while s >= 1:
        left = jnp.roll(x, -s, axis=1)
        right = jnp.roll(x, s, axis=1)
        x = jnp.where((idx & s) == 0, left, right)
        s //= 2
    return x


def _merge_sorted_desc(a, b):
    c = jnp.concatenate([a, _rev_lanes(b)], axis=1)
    idx = lax.broadcasted_iota(jnp.int32, c.shape, 1)
    s = K
    while s >= 1:
        left = jnp.roll(c, -s, axis=1)
        right = jnp.roll(c, s, axis=1)
        up = (idx & s) == 0
        partner = jnp.where(up, left, right)
        c = jnp.where(up, jnp.maximum(c, partner), jnp.minimum(c, partner))
        s //= 2
    return c[:, :K]


def _topk_desc(vals_list, n_rows, k):
    out = jnp.full((n_rows, k), NEG_INF, dtype=jnp.float32)
    col = lax.broadcasted_iota(jnp.int32, (n_rows, k), 1)
    arrs = list(vals_list)
    for i in range(k):
        m = arrs[0].max(axis=1, keepdims=True)
        for a in arrs[1:]:
            m = jnp.maximum(m, a.max(axis=1, keepdims=True))
        out = jnp.where(col == i, m, out)
        arrs = [jnp.where(a == m, NEG_INF, a) for a in arrs]
    return out


def _chunk_topk(xv):
    x = xv
    cands = []
    for t in range(CAND_T):
        a = x
        w = N_LOCAL
        while w > N_GROUP:
            w //= 2
            a = jnp.maximum(a[:, :w], a[:, w:])
        cands.append(a)
        if t < CAND_T - 1:
            b = jnp.concatenate([a] * (N_LOCAL // N_GROUP), axis=1)
            x = jnp.where(x == b, NEG_INF, x)

    F = cands[0]
    d = jnp.zeros(F.shape, jnp.int32)
    out = jnp.full((ROW_BLK, K), NEG_INF, dtype=jnp.float32)
    col = lax.broadcasted_iota(jnp.int32, (ROW_BLK, K), 1)
    for i in range(K):
        m = F.max(axis=1, keepdims=True)
        out = jnp.where(col == i, m, out)
        adv = F == m
        d = d + adv.astype(jnp.int32)
        nxt = jnp.full_like(F, NEG_INF)
        for t in range(CAND_T - 1, 0, -1):
            nxt = jnp.where(d == t, cands[t], nxt)
        F = jnp.where(adv, nxt, F)
    return out


def kernel(x):
    def body(x_hbm, o_ref, xv_ref, comm_ref, dma_sem,
             ex_send, ex_recv, ga_send, ga_recv):
        mx = lax.axis_index("x")
        my = lax.axis_index("y")
        mz = lax.axis_index("z")
        g = mx * 4 + mz
        partner = (mx, 1 - my, mz)

        cp = pltpu.make_async_copy(
            x_hbm.at[pl.ds(g * ROW_BLK, ROW_BLK), :], xv_ref, dma_sem
        )
        cp.start()

        bsem = pltpu.get_barrier_semaphore()
        pl.semaphore_signal(bsem, inc=1, device_id=partner,
                            device_id_type=pl.DeviceIdType.MESH)
        for p in range(N_GRP):
            px, pz = p // 4, p % 4

            @pl.when(g != p)
            def _():
                pl.semaphore_signal(bsem, inc=1, device_id=(px, my, pz),
                                    device_id_type=pl.DeviceIdType.MESH)

        pl.semaphore_wait(bsem, N_GRP)
        cp.wait()

        comm_ref[0] = _chunk_topk(xv_ref[...])

        rdma = pltpu.make_async_remote_copy(
            src_ref=comm_ref.at[0],
            dst_ref=comm_ref.at[1],
            send_sem=ex_send,
            recv_sem=ex_recv,
            device_id=partner,
            device_id_type=pl.DeviceIdType.MESH,
        )
        rdma.start()
        rdma.wait()

        o_ref[pl.ds(g * ROW_BLK, ROW_BLK), :] = _merge_sorted_desc(
            comm_ref[0], comm_ref[1]
        )

        for p in range(N_GRP):
            px, pz = p // 4, p % 4

            @pl.when(g != p)
            def _():
                send = pltpu.make_async_remote_copy(
                    src_ref=o_ref.at[pl.ds(g * ROW_BLK, ROW_BLK), :],
                    dst_ref=o_ref.at[pl.ds(g * ROW_BLK, ROW_BLK), :],
                    send_sem=ga_send.at[p],
                    recv_sem=ga_recv.at[g],
                    device_id=(px, my, pz),
                    device_id_type=pl.DeviceIdType.MESH,
                )
                send.start()

        for p in range(N_GRP):
            px, pz = p // 4, p % 4

            @pl.when(g != p)
            def _():
                recv = pltpu.make_async_remote_copy(
                    src_ref=o_ref.at[pl.ds(p * ROW_BLK, ROW_BLK), :],
                    dst_ref=o_ref.at[pl.ds(p * ROW_BLK, ROW_BLK), :],
                    send_sem=ga_send.at[p],
                    recv_sem=ga_recv.at[p],
                    device_id=(px, my, pz),
                    device_id_type=pl.DeviceIdType.MESH,
                )
                recv.wait_recv()

        for p in range(N_GRP):
            px, pz = p // 4, p % 4

            @pl.when(g != p)
            def _():
                snt = pltpu.make_async_remote_copy(
                    src_ref=o_ref.at[pl.ds(g * ROW_BLK, ROW_BLK), :],
                    dst_ref=o_ref.at[pl.ds(g * ROW_BLK, ROW_BLK), :],
                    send_sem=ga_send.at[p],
                    recv_sem=ga_recv.at[g],
                    device_id=(px, my, pz),
                    device_id_type=pl.DeviceIdType.MESH,
                )
                snt.wait_send()

    return pl.pallas_call(
        body,
        out_shape=jax.ShapeDtypeStruct((M, K), jnp.float32),
        in_specs=[pl.BlockSpec(memory_space=pl.ANY)],
        out_specs=pl.BlockSpec(memory_space=pltpu.VMEM),
        scratch_shapes=[
            pltpu.VMEM((ROW_BLK, N_LOCAL), jnp.float32),
            pltpu.VMEM((2, ROW_BLK, K), jnp.float32),
            pltpu.SemaphoreType.DMA,
            pltpu.SemaphoreType.DMA,
            pltpu.SemaphoreType.DMA,
            pltpu.SemaphoreType.DMA((N_GRP,)),
            pltpu.SemaphoreType.DMA((N_GRP,)),
        ],
        compiler_params=pltpu.CompilerParams(
            collective_id=0, vmem_limit_bytes=100 * 1024 * 1024
        ),
    )(x)
